# Initial kernel scaffold; baseline (speedup 1.0000x reference)
import jax
import jax.numpy as jnp
from jax.experimental import pallas as pl


def kernel(active_mask, edge_index, node_types, operator_indices, type_emb, act_emb, W1, att_src1, att_dst1, b1, ln1_g, ln1_b, W2, att_src2, att_dst2, b2, ln2_g, ln2_b, rW1, rb1, rW2, rb2, ln3_g, ln3_b):
    raise NotImplementedError("write your pallas kernel here")


# speedup vs baseline: 9.1484x; 9.1484x over previous
import functools

import jax
import jax.numpy as jnp
from jax import lax
from jax.experimental import pallas as pl
from jax.experimental.pallas import tpu as pltpu
from jax.experimental.pallas import tpu_sc as plsc

N = 10000
E = 160000
H = 4
D = 256
HD = H * D
NG = 8
GW = 128
NOP = 2048
ET = E + N
NC = 2
NS = 16
NW = NC * NS
SUB = 128
NSUB = 42
EC = SUB * NSUB
ETP = EC * NW
BN = 80
NB = N // BN
NP = 10240
RPT = NP // NS

_f32 = jnp.float32
_i32 = jnp.int32


def _tail_matmul(xb, W_ref, asf_ref, adf_ref, hg_ref, ac_ref):
    hb = jnp.dot(xb, W_ref[...], preferred_element_type=_f32)
    for g in range(NG):
        hg_ref[g] = hb[:, g * GW:(g + 1) * GW]
    cols = []
    for hh in range(H):
        sl = hb[:, hh * D:(hh + 1) * D]
        cols.append(jnp.sum(sl * asf_ref[0, hh * D:(hh + 1) * D][None, :],
                            axis=1, keepdims=True))
    for hh in range(H):
        sl = hb[:, hh * D:(hh + 1) * D]
        cols.append(jnp.sum(sl * adf_ref[0, hh * D:(hh + 1) * D][None, :],
                            axis=1, keepdims=True))
    ac_ref[...] = jnp.concatenate(cols, axis=1)


def _tc1_body(nt_ref, am_ref, T_ref, W_ref, asf_ref, adf_ref,
              x_ref, hg_ref, ac_ref):
    col = lax.broadcasted_iota(_i32, (BN, 128), 1)
    oh = (col == nt_ref[...]).astype(_f32)
    oh = oh + (col == (am_ref[...].astype(_i32) + 3)).astype(_f32)
    xb = jnp.dot(oh, T_ref[...], preferred_element_type=_f32)
    x_ref[...] = xb
    _tail_matmul(xb, W_ref, asf_ref, adf_ref, hg_ref, ac_ref)


def _mix_layer(xprev, og_ref, b_ref, g_ref, be_ref):
    og = og_ref[0] + og_ref[1]
    m0 = (og[0] + og[2] + og[4] + og[6]) * 0.25
    m1 = (og[1] + og[3] + og[5] + og[7]) * 0.25
    hmean = jnp.concatenate([m0, m1], axis=1) + b_ref[...]
    mu = jnp.mean(hmean, axis=1, keepdims=True)
    var = jnp.mean((hmean - mu) ** 2, axis=1, keepdims=True)
    ln = (hmean - mu) * lax.rsqrt(var + 1e-5) * g_ref[...] + be_ref[...]
    return xprev + jnp.maximum(ln, 0.0)


def _tc2_body(xp_ref, og_ref, b_ref, g_ref, be_ref, W_ref, asf_ref, adf_ref,
              x_ref, hg_ref, ac_ref):
    xb = _mix_layer(xp_ref[...], og_ref, b_ref, g_ref, be_ref)
    x_ref[...] = xb
    _tail_matmul(xb, W_ref, asf_ref, adf_ref, hg_ref, ac_ref)


def _tc3_body(xp_ref, og_ref, b_ref, g_ref, be_ref, am_ref,
              x_ref, n2_ref, gacc_ref, macc_ref):
    i = pl.program_id(0)
    xb = _mix_layer(xp_ref[...], og_ref, b_ref, g_ref, be_ref)
    x_ref[...] = xb
    n2_ref[...] = jnp.sum(xb * xb, axis=1, keepdims=True)
    amb = am_ref[...]

    @pl.when(i == 0)
    def _():
        gacc_ref[...] = jnp.zeros_like(gacc_ref)
        macc_ref[...] = jnp.zeros_like(macc_ref)

    gacc_ref[...] += jnp.sum(xb * amb, axis=0, keepdims=True)
    macc_ref[...] += jnp.sum(amb, axis=0, keepdims=True)


def _tc4_body(gacc_ref, macc_ref, ss_ref, rW1_ref, rb1_ref, rW2_ref, rb2_ref,
              g3_ref, b3_ref, hl_ref, os_ref):
    g = gacc_ref[...] / (macc_ref[0, 0] + 1e-8)
    hid = jnp.maximum(
        jnp.dot(g, rW1_ref[...], preferred_element_type=_f32) + rb1_ref[...],
        0.0)
    hid = jnp.dot(hid, rW2_ref[...], preferred_element_type=_f32) + rb2_ref[...]
    mu = jnp.mean(hid, axis=1, keepdims=True)
    var = jnp.mean((hid - mu) ** 2, axis=1, keepdims=True)
    hl_ref[...] = ((hid - mu) * lax.rsqrt(var + 1e-5) * g3_ref[...]
                   + b3_ref[...])
    os_ref[...] = jnp.sqrt(ss_ref[...])


def _tc_layer1(nt2, am2, T, W1, asf, adf):
    full = lambda s: pl.BlockSpec(s, lambda i: (0,) * len(s))
    return pl.pallas_call(
        _tc1_body,
        grid=(NB,),
        in_specs=[
            pl.BlockSpec((BN, 1), lambda i: (i, 0)),
            pl.BlockSpec((BN, 1), lambda i: (i, 0)),
            full((128, D)), full((D, HD)), full((1, HD)), full((1, HD)),
        ],
        out_specs=[
            pl.BlockSpec((BN, D), lambda i: (i, 0)),
            pl.BlockSpec((NG, BN, GW), lambda i: (0, i, 0)),
            pl.BlockSpec((BN, 2 * H), lambda i: (i, 0)),
        ],
        out_shape=[
            jax.ShapeDtypeStruct((N, D), _f32),
            jax.ShapeDtypeStruct((NG, N, GW), _f32),
            jax.ShapeDtypeStruct((N, 2 * H), _f32),
        ],
    )(nt2, am2, T, W1, asf, adf)


def _tc_layer2(xp, og, b, g, be, W2, asf, adf):
    full = lambda s: pl.BlockSpec(s, lambda i: (0,) * len(s))
    return pl.pallas_call(
        _tc2_body,
        grid=(NB,),
        in_specs=[
            pl.BlockSpec((BN, D), lambda i: (i, 0)),
            pl.BlockSpec((NC, NG, BN, GW), lambda i: (0, 0, i, 0)),
            full((1, D)), full((1, D)), full((1, D)),
            full((D, HD)), full((1, HD)), full((1, HD)),
        ],
        out_specs=[
            pl.BlockSpec((BN, D), lambda i: (i, 0)),
            pl.BlockSpec((NG, BN, GW), lambda i: (0, i, 0)),
            pl.BlockSpec((BN, 2 * H), lambda i: (i, 0)),
        ],
        out_shape=[
            jax.ShapeDtypeStruct((N, D), _f32),
            jax.ShapeDtypeStruct((NG, N, GW), _f32),
            jax.ShapeDtypeStruct((N, 2 * H), _f32),
        ],
    )(xp, og, b, g, be, W2, asf, adf)


def _tc_final(xp, og, b, g, be, am2):
    full = lambda s: pl.BlockSpec(s, lambda i: (0,) * len(s))
    return pl.pallas_call(
        _tc3_body,
        grid=(NB,),
        in_specs=[
            pl.BlockSpec((BN, D), lambda i: (i, 0)),
            pl.BlockSpec((NC, NG, BN, GW), lambda i: (0, 0, i, 0)),
            full((1, D)), full((1, D)), full((1, D)),
            pl.BlockSpec((BN, 1), lambda i: (i, 0)),
        ],
        out_specs=[
            pl.BlockSpec((BN, D), lambda i: (i, 0)),
            pl.BlockSpec((BN, 1), lambda i: (i, 0)),
            pl.BlockSpec((1, D), lambda i: (0, 0)),
            pl.BlockSpec((1, 1), lambda i: (0, 0)),
        ],
        out_shape=[
            jax.ShapeDtypeStruct((N, D), _f32),
            jax.ShapeDtypeStruct((N, 1), _f32),
            jax.ShapeDtypeStruct((1, D), _f32),
            jax.ShapeDtypeStruct((1, 1), _f32),
        ],
    )(xp, og, b, g, be, am2)


def _tc_head(gacc, macc, ss2, rW1, rb1, rW2, rb2, g3, b3):
    full = lambda s: pl.BlockSpec(s, lambda: (0,) * len(s))
    return pl.pallas_call(
        _tc4_body,
        in_specs=[
            full((1, D)), full((1, 1)), full((NOP // 128, 128)),
            full((D, D)), full((1, D)), full((D, D)), full((1, D)),
            full((1, D)), full((1, D)),
        ],
        out_specs=[full((1, D)), full((NOP // 128, 128))],
        out_shape=[
            jax.ShapeDtypeStruct((1, D), _f32),
            jax.ShapeDtypeStruct((NOP // 128, 128), _f32),
        ],
    )(gacc, macc, ss2, rW1, rb1, rW2, rb2, g3, b3)


_MESH = dict(core_axis_name="c", subcore_axis_name="s")
_SC_PARAMS = dict(
    mesh=plsc.VectorSubcoreMesh(**_MESH),
    compiler_params=pltpu.CompilerParams(needs_layout_passes=False),
)


def _sc_pass1(attc_f, src3, dst3, z1):
    @functools.partial(
        pl.kernel,
        out_type=[
            jax.ShapeDtypeStruct((NW, NSUB, H, SUB), _f32),
            jax.ShapeDtypeStruct((NC, H, NP), _f32),
        ],
        scratch_types=[
            pltpu.VMEM((N * 2 * H,), _f32),
            pltpu.VMEM((NSUB, SUB), _i32),
            pltpu.VMEM((NSUB, SUB), _i32),
            pltpu.VMEM((H, SUB), _f32),
            pltpu.VMEM_SHARED((NP,), _f32),
            pltpu.VMEM_SHARED((NP,), _f32),
            pltpu.VMEM_SHARED((NP,), _f32),
            pltpu.VMEM_SHARED((NP,), _f32),
        ],
        **_SC_PARAMS,
    )
    def k(attc_hbm, src_hbm, dst_hbm, z1_hbm, ex_hbm, dpart_hbm,
          att_v, src_v, dst_v, ex_v, den0, den1, den2, den3):
        dens = (den0, den1, den2, den3)
        cid = lax.axis_index("c")
        sid = lax.axis_index("s")
        wid = sid * NC + cid
        pltpu.sync_copy(attc_hbm, att_v)
        pltpu.sync_copy(src_hbm.at[wid], src_v)
        pltpu.sync_copy(dst_hbm.at[wid], dst_v)
        r0 = sid * RPT
        for hh in range(H):
            pltpu.sync_copy(z1_hbm.at[pl.ds(r0, RPT)],
                            dens[hh].at[pl.ds(r0, RPT)])
        plsc.subcore_barrier()
        lanes = lax.iota(_i32, 16)

        def sub(jj, carry):
            for k8 in range(SUB // 16):
                sv = src_v[jj, pl.ds(k8 * 16, 16)]
                dv = dst_v[jj, pl.ds(k8 * 16, 16)]
                gid = wid * EC + jj * SUB + k8 * 16 + lanes
                valid = gid < ET
                for hh in range(H):
                    a1 = plsc.load_gather(att_v, [sv * (2 * H) + hh])
                    a2 = plsc.load_gather(att_v, [dv * (2 * H) + (H + hh)])
                    al = a1 + a2
                    al = jnp.where(al >= 0.0, al, 0.2 * al)
                    exv = jnp.where(valid, jnp.exp(al), 0.0)
                    ex_v[hh, pl.ds(k8 * 16, 16)] = exv
            pltpu.sync_copy(ex_v, ex_hbm.at[wid, jj])
            for hh in range(H):
                pltpu.sync_copy(ex_v.at[hh], dens[hh].at[dst_v.at[jj]],
                                add=True)
            return carry

        lax.fori_loop(0, NSUB, sub, 0)
        plsc.subcore_barrier()
        for hh in range(H):
            pltpu.sync_copy(dens[hh].at[pl.ds(r0, RPT)],
                            dpart_hbm.at[cid, hh, pl.ds(r0, RPT)])

    return k(attc_f, src3, dst3, z1)


def _sc_pass2(exf, dpart2, dstf):
    @functools.partial(
        pl.kernel,
        out_type=jax.ShapeDtypeStruct((ETP * H,), _f32),
        scratch_types=[
            pltpu.VMEM((H * NP,), _f32),
            pltpu.VMEM((H * NP,), _f32),
            pltpu.VMEM((EC,), _i32),
            pltpu.VMEM((H * SUB,), _f32),
            pltpu.VMEM((H * SUB,), _f32),
        ],
        **_SC_PARAMS,
    )
    def k(ex_hbm, dpart_hbm, dst_hbm, attn_hbm, dA, dB, dst_f, ex_v, at_v):
        cid = lax.axis_index("c")
        sid = lax.axis_index("s")
        wid = sid * NC + cid
        pltpu.sync_copy(dpart_hbm.at[0], dA)
        pltpu.sync_copy(dpart_hbm.at[1], dB)
        pltpu.sync_copy(dst_hbm.at[pl.ds(wid * EC, EC)], dst_f)
        lanes = lax.iota(_i32, 16)
        lq = lanes // 4
        lr = lanes - lq * 4

        def sub(jj, carry):
            pltpu.sync_copy(ex_hbm.at[pl.ds((wid * NSUB + jj) * H * SUB,
                                            H * SUB)], ex_v)
            for grp in range(SUB * H // 16):
                e0 = grp * 4
                dv = plsc.load_gather(dst_f, [jj * SUB + e0 + lq])
                di = lr * NP + dv
                den = plsc.load_gather(dA, [di]) + plsc.load_gather(dB, [di])
                exg = plsc.load_gather(ex_v, [lr * SUB + e0 + lq])
                at_v[pl.ds(grp * 16, 16)] = exg / (den + 1e-16)
            pltpu.sync_copy(
                at_v, attn_hbm.at[pl.ds((wid * EC + jj * SUB) * H, H * SUB)])
            return carry

        lax.fori_loop(0, NSUB, sub, 0)

    return k(exf, dpart2, dstf)


def _sc_pass3(hflat, attnf, srcg4, dst3, z128):
    @functools.partial(
        pl.kernel,
        out_type=jax.ShapeDtypeStruct((NC, NG, NP, GW), _f32),
        scratch_types=[
            pltpu.VMEM((NSUB, SUB), _i32),
            pltpu.VMEM((NSUB, SUB), _i32),
            pltpu.VMEM((H * SUB,), _f32),
            pltpu.VMEM((SUB, GW), _f32),
            pltpu.VMEM_SHARED((NP, GW), _f32),
            pltpu.SemaphoreType.DMA,
        ],
        **_SC_PARAMS,
    )
    def k(h_hbm, attn_hbm, srcg_hbm, dst_hbm, z_hbm, og_hbm,
          src_v, dst_v, at_v, rbuf, acc_sp, sem):
        cid = lax.axis_index("c")
        sid = lax.axis_index("s")
        wid = sid * NC + cid
        pltpu.sync_copy(dst_hbm.at[wid], dst_v)
        r0 = sid * RPT

        for g in range(NG):
            hcol = g // 2
            pltpu.sync_copy(srcg_hbm.at[g, wid], src_v)
            pltpu.sync_copy(z_hbm.at[pl.ds(r0, RPT)],
                            acc_sp.at[pl.ds(r0, RPT)])
            plsc.subcore_barrier()

            def sub(jj, carry):
                pltpu.sync_copy(
                    attn_hbm.at[pl.ds((wid * EC + jj * SUB) * H, H * SUB)],
                    at_v)
                pltpu.async_copy(h_hbm.at[src_v.at[jj]], rbuf, sem).wait()

                def scale16(j8, c2):
                    for e in range(16):
                        eg = j8 * 16 + e
                        sp = plsc.load_gather(
                            at_v, [jnp.broadcast_to(eg * H + hcol, (16,))])
                        for q in range(GW // 16):
                            rbuf[eg, pl.ds(q * 16, 16)] = (
                                rbuf[eg, pl.ds(q * 16, 16)] * sp)
                    return c2

                lax.fori_loop(0, SUB // 16, scale16, 0)
                pltpu.sync_copy(rbuf, acc_sp.at[dst_v.at[jj]], add=True)
                return carry

            lax.fori_loop(0, NSUB, sub, 0)
            plsc.subcore_barrier()
            pltpu.sync_copy(acc_sp.at[pl.ds(r0, RPT)],
                            og_hbm.at[cid, g, pl.ds(r0, RPT)])
            plsc.subcore_barrier()

    return k(hflat, attnf, srcg4, dst3, z128)


def _sc_opnorm(n2, ops2):
    @functools.partial(
        pl.kernel,
        out_type=jax.ShapeDtypeStruct((NOP,), _f32),
        scratch_types=[
            pltpu.VMEM((N,), _f32),
            pltpu.VMEM((NOP // NW,), _i32),
            pltpu.VMEM((NOP // NW,), _f32),
        ],
        **_SC_PARAMS,
    )
    def k(n2_hbm, ops_hbm, ss_hbm, n2_v, idx_v, s_v):
        cid = lax.axis_index("c")
        sid = lax.axis_index("s")
        wid = sid * NC + cid
        npt = NOP // NW
        pltpu.sync_copy(n2_hbm, n2_v)
        pltpu.sync_copy(ops_hbm.at[wid], idx_v)
        for grp in range(npt // 16):
            iv = idx_v[pl.ds(grp * 16, 16)]
            s_v[pl.ds(grp * 16, 16)] = plsc.load_gather(n2_v, [iv])
        pltpu.sync_copy(s_v, ss_hbm.at[pl.ds(wid * npt, npt)])

    return k(n2, ops2)


def kernel(active_mask, edge_index, node_types, operator_indices, type_emb,
           act_emb, W1, att_src1, att_dst1, b1, ln1_g, ln1_b, W2, att_src2,
           att_dst2, b2, ln2_g, ln2_b, rW1, rb1, rW2, rb2, ln3_g, ln3_b):
    loop = jnp.arange(N, dtype=edge_index.dtype)
    src = jnp.concatenate([edge_index[0], loop])
    dst = jnp.concatenate([edge_index[1], loop])
    pad = jnp.zeros((ETP - ET,), edge_index.dtype)
    srcp = jnp.concatenate([src, pad]).astype(_i32)
    dstp = jnp.concatenate([dst, pad]).astype(_i32)
    src3 = srcp.reshape(NW, NSUB, SUB)
    dst3 = dstp.reshape(NW, NSUB, SUB)
    srcg4 = src3[None] + jnp.arange(NG, dtype=_i32)[:, None, None, None] * N
    T = jnp.concatenate([type_emb, act_emb,
                         jnp.zeros((128 - 5, D), _f32)], axis=0)
    nt2 = node_types.astype(_i32).reshape(N, 1)
    am2 = active_mask.reshape(N, 1)
    z1 = jnp.zeros((NP,), _f32)
    z128 = jnp.zeros((NP, GW), _f32)
    ops2 = operator_indices.astype(_i32).reshape(NW, NOP // NW)

    x, hg1, ac1 = _tc_layer1(nt2, am2, T, W1,
                             att_src1.reshape(1, HD), att_dst1.reshape(1, HD))
    ex1, dp1 = _sc_pass1(ac1.reshape(N * 2 * H), src3, dst3, z1)
    attn1 = _sc_pass2(ex1.reshape(ETP * H), dp1.reshape(NC, H * NP), dstp)
    og1 = _sc_pass3(hg1.reshape(NG * N, GW), attn1, srcg4, dst3, z128)

    x1, hg2, ac2 = _tc_layer2(x, og1, b1.reshape(1, D), ln1_g.reshape(1, D),
                              ln1_b.reshape(1, D), W2,
                              att_src2.reshape(1, HD), att_dst2.reshape(1, HD))
    ex2, dp2 = _sc_pass1(ac2.reshape(N * 2 * H), src3, dst3, z1)
    attn2 = _sc_pass2(ex2.reshape(ETP * H), dp2.reshape(NC, H * NP), dstp)
    og2 = _sc_pass3(hg2.reshape(NG * N, GW), attn2, srcg4, dst3, z128)

    x2, n2, gacc, macc = _tc_final(x1, og2, b2.reshape(1, D),
                                   ln2_g.reshape(1, D), ln2_b.reshape(1, D),
                                   am2)
    del x2
    ss = _sc_opnorm(n2.reshape(N), ops2)
    hl, osc = _tc_head(gacc, macc, ss.reshape(NOP // 128, 128),
                       rW1, rb1.reshape(1, D), rW2, rb2.reshape(1, D),
                       ln3_g.reshape(1, D), ln3_b.reshape(1, D))
    return hl.reshape(D), osc.reshape(NOP), attn2.reshape(ETP, H)[:ET]


# speedup vs baseline: 12.0757x; 1.3200x over previous
import functools

import jax
import jax.numpy as jnp
from jax import lax
from jax.experimental import pallas as pl
from jax.experimental.pallas import tpu as pltpu
from jax.experimental.pallas import tpu_sc as plsc

N = 10000
E = 160000
H = 4
D = 256
HD = H * D
NG = 8
GW = 128
NOP = 2048
ET = E + N
NC = 2
NS = 16
NW = NC * NS
SUB = 128
NSUB = 42
EC = SUB * NSUB
ETP = EC * NW
BN = 80
NB = N // BN
NP = 10240
RPT = NP // NS

_f32 = jnp.float32
_i32 = jnp.int32


def _tail_matmul(xb, W_ref, asf_ref, adf_ref, hg_ref, ac_ref):
    hb = jnp.dot(xb, W_ref[...], preferred_element_type=_f32)
    for g in range(NG):
        hg_ref[g] = hb[:, g * GW:(g + 1) * GW]
    cols = []
    for hh in range(H):
        sl = hb[:, hh * D:(hh + 1) * D]
        cols.append(jnp.sum(sl * asf_ref[0, hh * D:(hh + 1) * D][None, :],
                            axis=1, keepdims=True))
    for hh in range(H):
        sl = hb[:, hh * D:(hh + 1) * D]
        cols.append(jnp.sum(sl * adf_ref[0, hh * D:(hh + 1) * D][None, :],
                            axis=1, keepdims=True))
    ac_ref[...] = jnp.concatenate(cols, axis=1)


def _tc1_body(nt_ref, am_ref, T_ref, W_ref, asf_ref, adf_ref,
              x_ref, hg_ref, ac_ref):
    col = lax.broadcasted_iota(_i32, (BN, 128), 1)
    oh = (col == nt_ref[...]).astype(_f32)
    oh = oh + (col == (am_ref[...].astype(_i32) + 3)).astype(_f32)
    xb = jnp.dot(oh, T_ref[...], preferred_element_type=_f32)
    x_ref[...] = xb
    _tail_matmul(xb, W_ref, asf_ref, adf_ref, hg_ref, ac_ref)


def _mix_layer(xprev, og_ref, b_ref, g_ref, be_ref):
    og = og_ref[0] + og_ref[1]
    m0 = (og[0] + og[2] + og[4] + og[6]) * 0.25
    m1 = (og[1] + og[3] + og[5] + og[7]) * 0.25
    hmean = jnp.concatenate([m0, m1], axis=1) + b_ref[...]
    mu = jnp.mean(hmean, axis=1, keepdims=True)
    var = jnp.mean((hmean - mu) ** 2, axis=1, keepdims=True)
    ln = (hmean - mu) * lax.rsqrt(var + 1e-5) * g_ref[...] + be_ref[...]
    return xprev + jnp.maximum(ln, 0.0)


def _tc2_body(xp_ref, og_ref, b_ref, g_ref, be_ref, W_ref, asf_ref, adf_ref,
              x_ref, hg_ref, ac_ref):
    xb = _mix_layer(xp_ref[...], og_ref, b_ref, g_ref, be_ref)
    x_ref[...] = xb
    _tail_matmul(xb, W_ref, asf_ref, adf_ref, hg_ref, ac_ref)


def _tc3_body(xp_ref, og_ref, b_ref, g_ref, be_ref, am_ref,
              x_ref, n2_ref, gacc_ref, macc_ref):
    i = pl.program_id(0)
    xb = _mix_layer(xp_ref[...], og_ref, b_ref, g_ref, be_ref)
    x_ref[...] = xb
    n2_ref[...] = jnp.sum(xb * xb, axis=1, keepdims=True)
    amb = am_ref[...]

    @pl.when(i == 0)
    def _():
        gacc_ref[...] = jnp.zeros_like(gacc_ref)
        macc_ref[...] = jnp.zeros_like(macc_ref)

    gacc_ref[...] += jnp.sum(xb * amb, axis=0, keepdims=True)
    macc_ref[...] += jnp.sum(amb, axis=0, keepdims=True)


def _tc4_body(gacc_ref, macc_ref, ss_ref, rW1_ref, rb1_ref, rW2_ref, rb2_ref,
              g3_ref, b3_ref, hl_ref, os_ref):
    g = gacc_ref[...] / (macc_ref[0, 0] + 1e-8)
    hid = jnp.maximum(
        jnp.dot(g, rW1_ref[...], preferred_element_type=_f32) + rb1_ref[...],
        0.0)
    hid = jnp.dot(hid, rW2_ref[...], preferred_element_type=_f32) + rb2_ref[...]
    mu = jnp.mean(hid, axis=1, keepdims=True)
    var = jnp.mean((hid - mu) ** 2, axis=1, keepdims=True)
    hl_ref[...] = ((hid - mu) * lax.rsqrt(var + 1e-5) * g3_ref[...]
                   + b3_ref[...])
    os_ref[...] = jnp.sqrt(ss_ref[...])


def _tc_layer1(nt2, am2, T, W1, asf, adf):
    full = lambda s: pl.BlockSpec(s, lambda i: (0,) * len(s))
    return pl.pallas_call(
        _tc1_body,
        grid=(NB,),
        in_specs=[
            pl.BlockSpec((BN, 1), lambda i: (i, 0)),
            pl.BlockSpec((BN, 1), lambda i: (i, 0)),
            full((128, D)), full((D, HD)), full((1, HD)), full((1, HD)),
        ],
        out_specs=[
            pl.BlockSpec((BN, D), lambda i: (i, 0)),
            pl.BlockSpec((NG, BN, GW), lambda i: (0, i, 0)),
            pl.BlockSpec((BN, 2 * H), lambda i: (i, 0)),
        ],
        out_shape=[
            jax.ShapeDtypeStruct((N, D), _f32),
            jax.ShapeDtypeStruct((NG, N, GW), _f32),
            jax.ShapeDtypeStruct((N, 2 * H), _f32),
        ],
    )(nt2, am2, T, W1, asf, adf)


def _tc_layer2(xp, og, b, g, be, W2, asf, adf):
    full = lambda s: pl.BlockSpec(s, lambda i: (0,) * len(s))
    return pl.pallas_call(
        _tc2_body,
        grid=(NB,),
        in_specs=[
            pl.BlockSpec((BN, D), lambda i: (i, 0)),
            pl.BlockSpec((NC, NG, BN, GW), lambda i: (0, 0, i, 0)),
            full((1, D)), full((1, D)), full((1, D)),
            full((D, HD)), full((1, HD)), full((1, HD)),
        ],
        out_specs=[
            pl.BlockSpec((BN, D), lambda i: (i, 0)),
            pl.BlockSpec((NG, BN, GW), lambda i: (0, i, 0)),
            pl.BlockSpec((BN, 2 * H), lambda i: (i, 0)),
        ],
        out_shape=[
            jax.ShapeDtypeStruct((N, D), _f32),
            jax.ShapeDtypeStruct((NG, N, GW), _f32),
            jax.ShapeDtypeStruct((N, 2 * H), _f32),
        ],
    )(xp, og, b, g, be, W2, asf, adf)


def _tc_final(xp, og, b, g, be, am2):
    full = lambda s: pl.BlockSpec(s, lambda i: (0,) * len(s))
    return pl.pallas_call(
        _tc3_body,
        grid=(NB,),
        in_specs=[
            pl.BlockSpec((BN, D), lambda i: (i, 0)),
            pl.BlockSpec((NC, NG, BN, GW), lambda i: (0, 0, i, 0)),
            full((1, D)), full((1, D)), full((1, D)),
            pl.BlockSpec((BN, 1), lambda i: (i, 0)),
        ],
        out_specs=[
            pl.BlockSpec((BN, D), lambda i: (i, 0)),
            pl.BlockSpec((BN, 1), lambda i: (i, 0)),
            pl.BlockSpec((1, D), lambda i: (0, 0)),
            pl.BlockSpec((1, 1), lambda i: (0, 0)),
        ],
        out_shape=[
            jax.ShapeDtypeStruct((N, D), _f32),
            jax.ShapeDtypeStruct((N, 1), _f32),
            jax.ShapeDtypeStruct((1, D), _f32),
            jax.ShapeDtypeStruct((1, 1), _f32),
        ],
    )(xp, og, b, g, be, am2)


def _tc_head(gacc, macc, ss2, rW1, rb1, rW2, rb2, g3, b3):
    full = lambda s: pl.BlockSpec(s, lambda: (0,) * len(s))
    return pl.pallas_call(
        _tc4_body,
        in_specs=[
            full((1, D)), full((1, 1)), full((NOP // 128, 128)),
            full((D, D)), full((1, D)), full((D, D)), full((1, D)),
            full((1, D)), full((1, D)),
        ],
        out_specs=[full((1, D)), full((NOP // 128, 128))],
        out_shape=[
            jax.ShapeDtypeStruct((1, D), _f32),
            jax.ShapeDtypeStruct((NOP // 128, 128), _f32),
        ],
    )(gacc, macc, ss2, rW1, rb1, rW2, rb2, g3, b3)


_MESH = dict(core_axis_name="c", subcore_axis_name="s")
_SC_PARAMS = dict(
    mesh=plsc.VectorSubcoreMesh(**_MESH),
    compiler_params=pltpu.CompilerParams(needs_layout_passes=False),
)


def _sc_pass1(attc_f, src3, dst3, z1):
    @functools.partial(
        pl.kernel,
        out_type=[
            jax.ShapeDtypeStruct((NW, NSUB, H, SUB), _f32),
            jax.ShapeDtypeStruct((NC, H, NP), _f32),
        ],
        scratch_types=[
            pltpu.VMEM((N * 2 * H,), _f32),
            pltpu.VMEM((NSUB, SUB), _i32),
            pltpu.VMEM((NSUB, SUB), _i32),
            pltpu.VMEM((H, SUB), _f32),
            pltpu.VMEM_SHARED((NP,), _f32),
            pltpu.VMEM_SHARED((NP,), _f32),
            pltpu.VMEM_SHARED((NP,), _f32),
            pltpu.VMEM_SHARED((NP,), _f32),
        ],
        **_SC_PARAMS,
    )
    def k(attc_hbm, src_hbm, dst_hbm, z1_hbm, ex_hbm, dpart_hbm,
          att_v, src_v, dst_v, ex_v, den0, den1, den2, den3):
        dens = (den0, den1, den2, den3)
        cid = lax.axis_index("c")
        sid = lax.axis_index("s")
        wid = sid * NC + cid
        pltpu.sync_copy(attc_hbm, att_v)
        pltpu.sync_copy(src_hbm.at[wid], src_v)
        pltpu.sync_copy(dst_hbm.at[wid], dst_v)
        r0 = sid * RPT
        for hh in range(H):
            pltpu.sync_copy(z1_hbm.at[pl.ds(r0, RPT)],
                            dens[hh].at[pl.ds(r0, RPT)])
        plsc.subcore_barrier()
        lanes = lax.iota(_i32, 16)

        def sub(jj, carry):
            for k8 in range(SUB // 16):
                sv = src_v[jj, pl.ds(k8 * 16, 16)]
                dv = dst_v[jj, pl.ds(k8 * 16, 16)]
                gid = wid * EC + jj * SUB + k8 * 16 + lanes
                valid = gid < ET
                for hh in range(H):
                    a1 = plsc.load_gather(att_v, [sv * (2 * H) + hh])
                    a2 = plsc.load_gather(att_v, [dv * (2 * H) + (H + hh)])
                    al = a1 + a2
                    al = jnp.where(al >= 0.0, al, 0.2 * al)
                    exv = jnp.where(valid, jnp.exp(al), 0.0)
                    ex_v[hh, pl.ds(k8 * 16, 16)] = exv
            pltpu.sync_copy(ex_v, ex_hbm.at[wid, jj])
            for hh in range(H):
                pltpu.sync_copy(ex_v.at[hh], dens[hh].at[dst_v.at[jj]],
                                add=True)
            return carry

        lax.fori_loop(0, NSUB, sub, 0)
        plsc.subcore_barrier()
        for hh in range(H):
            pltpu.sync_copy(dens[hh].at[pl.ds(r0, RPT)],
                            dpart_hbm.at[cid, hh, pl.ds(r0, RPT)])

    return k(attc_f, src3, dst3, z1)


def _sc_pass2(exf, dpart2, dstf):
    @functools.partial(
        pl.kernel,
        out_type=jax.ShapeDtypeStruct((ETP * H,), _f32),
        scratch_types=[
            pltpu.VMEM((H * NP,), _f32),
            pltpu.VMEM((H * NP,), _f32),
            pltpu.VMEM((EC,), _i32),
            pltpu.VMEM((H * SUB,), _f32),
            pltpu.VMEM((H * SUB,), _f32),
        ],
        **_SC_PARAMS,
    )
    def k(ex_hbm, dpart_hbm, dst_hbm, attn_hbm, dA, dB, dst_f, ex_v, at_v):
        cid = lax.axis_index("c")
        sid = lax.axis_index("s")
        wid = sid * NC + cid
        pltpu.sync_copy(dpart_hbm.at[0], dA)
        pltpu.sync_copy(dpart_hbm.at[1], dB)
        pltpu.sync_copy(dst_hbm.at[pl.ds(wid * EC, EC)], dst_f)
        lanes = lax.iota(_i32, 16)
        lq = lanes // 4
        lr = lanes - lq * 4

        def sub(jj, carry):
            pltpu.sync_copy(ex_hbm.at[pl.ds((wid * NSUB + jj) * H * SUB,
                                            H * SUB)], ex_v)
            for grp in range(SUB * H // 16):
                e0 = grp * 4
                dv = plsc.load_gather(dst_f, [jj * SUB + e0 + lq])
                di = lr * NP + dv
                den = plsc.load_gather(dA, [di]) + plsc.load_gather(dB, [di])
                exg = plsc.load_gather(ex_v, [lr * SUB + e0 + lq])
                at_v[pl.ds(grp * 16, 16)] = exg / (den + 1e-16)
            pltpu.sync_copy(
                at_v, attn_hbm.at[pl.ds((wid * EC + jj * SUB) * H, H * SUB)])
            return carry

        lax.fori_loop(0, NSUB, sub, 0)

    return k(exf, dpart2, dstf)


def _sc_pass3(hflat, attnf, srcg4, dst3, z128):
    @functools.partial(
        pl.kernel,
        out_type=jax.ShapeDtypeStruct((NC, NG, NP, GW), _f32),
        scratch_types=[
            pltpu.VMEM((NSUB, SUB), _i32),
            pltpu.VMEM((NSUB, SUB), _i32),
            pltpu.VMEM((H * SUB,), _f32),
            pltpu.VMEM((H * SUB,), _f32),
            pltpu.VMEM((SUB, GW), _f32),
            pltpu.VMEM((SUB, GW), _f32),
            pltpu.VMEM_SHARED((NP, GW), _f32),
            pltpu.SemaphoreType.DMA,
            pltpu.SemaphoreType.DMA,
            pltpu.SemaphoreType.DMA,
            pltpu.SemaphoreType.DMA,
        ],
        **_SC_PARAMS,
    )
    def k(h_hbm, attn_hbm, srcg_hbm, dst_hbm, z_hbm, og_hbm,
          src_v, dst_v, at0, at1, rb0, rb1, acc_sp, sg0, sg1, sa0, sa1):
        ats = (at0, at1)
        rbs = (rb0, rb1)
        sgs = (sg0, sg1)
        sas = (sa0, sa1)
        cid = lax.axis_index("c")
        sid = lax.axis_index("s")
        wid = sid * NC + cid
        pltpu.sync_copy(dst_hbm.at[wid], dst_v)
        r0 = sid * RPT

        def at_slice(jj):
            return attn_hbm.at[pl.ds((wid * EC + jj * SUB) * H, H * SUB)]

        for g in range(NG):
            hcol = g // 2
            pltpu.sync_copy(srcg_hbm.at[g, wid], src_v)
            pltpu.sync_copy(z_hbm.at[pl.ds(r0, RPT)],
                            acc_sp.at[pl.ds(r0, RPT)])
            plsc.subcore_barrier()
            pltpu.async_copy(h_hbm.at[src_v.at[0]], rbs[0], sgs[0])
            pltpu.async_copy(at_slice(0), ats[0], sas[0])

            def step(st, carry):
                for b in range(2):
                    jj = st * 2 + b
                    bn = 1 - b
                    pltpu.make_async_copy(
                        h_hbm.at[src_v.at[jj]], rbs[b], sgs[b]).wait()
                    pltpu.make_async_copy(at_slice(jj), ats[b], sas[b]).wait()

                    @pl.when(jj + 1 < NSUB)
                    def _():
                        pltpu.async_copy(
                            h_hbm.at[src_v.at[jj + 1]], rbs[bn], sgs[bn])
                        pltpu.async_copy(at_slice(jj + 1), ats[bn], sas[bn])

                    rbuf = rbs[b]
                    at_v = ats[b]

                    def scale16(j8, c2):
                        for e in range(16):
                            eg = j8 * 16 + e
                            sp = plsc.load_gather(
                                at_v,
                                [jnp.broadcast_to(eg * H + hcol, (16,))])
                            for q in range(GW // 16):
                                rbuf[eg, pl.ds(q * 16, 16)] = (
                                    rbuf[eg, pl.ds(q * 16, 16)] * sp)
                        return c2

                    lax.fori_loop(0, SUB // 16, scale16, 0)
                    pltpu.sync_copy(rbs[b], acc_sp.at[dst_v.at[jj]], add=True)
                return carry

            lax.fori_loop(0, NSUB // 2, step, 0)
            plsc.subcore_barrier()
            pltpu.sync_copy(acc_sp.at[pl.ds(r0, RPT)],
                            og_hbm.at[cid, g, pl.ds(r0, RPT)])
            plsc.subcore_barrier()

    return k(hflat, attnf, srcg4, dst3, z128)


def _sc_opnorm(n2, ops2):
    @functools.partial(
        pl.kernel,
        out_type=jax.ShapeDtypeStruct((NOP,), _f32),
        scratch_types=[
            pltpu.VMEM((N,), _f32),
            pltpu.VMEM((NOP // NW,), _i32),
            pltpu.VMEM((NOP // NW,), _f32),
        ],
        **_SC_PARAMS,
    )
    def k(n2_hbm, ops_hbm, ss_hbm, n2_v, idx_v, s_v):
        cid = lax.axis_index("c")
        sid = lax.axis_index("s")
        wid = sid * NC + cid
        npt = NOP // NW
        pltpu.sync_copy(n2_hbm, n2_v)
        pltpu.sync_copy(ops_hbm.at[wid], idx_v)
        for grp in range(npt // 16):
            iv = idx_v[pl.ds(grp * 16, 16)]
            s_v[pl.ds(grp * 16, 16)] = plsc.load_gather(n2_v, [iv])
        pltpu.sync_copy(s_v, ss_hbm.at[pl.ds(wid * npt, npt)])

    return k(n2, ops2)


def kernel(active_mask, edge_index, node_types, operator_indices, type_emb,
           act_emb, W1, att_src1, att_dst1, b1, ln1_g, ln1_b, W2, att_src2,
           att_dst2, b2, ln2_g, ln2_b, rW1, rb1, rW2, rb2, ln3_g, ln3_b):
    loop = jnp.arange(N, dtype=edge_index.dtype)
    src = jnp.concatenate([edge_index[0], loop])
    dst = jnp.concatenate([edge_index[1], loop])
    pad = jnp.zeros((ETP - ET,), edge_index.dtype)
    srcp = jnp.concatenate([src, pad]).astype(_i32)
    dstp = jnp.concatenate([dst, pad]).astype(_i32)
    src3 = srcp.reshape(NW, NSUB, SUB)
    dst3 = dstp.reshape(NW, NSUB, SUB)
    srcg4 = src3[None] + jnp.arange(NG, dtype=_i32)[:, None, None, None] * N
    T = jnp.concatenate([type_emb, act_emb,
                         jnp.zeros((128 - 5, D), _f32)], axis=0)
    nt2 = node_types.astype(_i32).reshape(N, 1)
    am2 = active_mask.reshape(N, 1)
    z1 = jnp.zeros((NP,), _f32)
    z128 = jnp.zeros((NP, GW), _f32)
    ops2 = operator_indices.astype(_i32).reshape(NW, NOP // NW)

    x, hg1, ac1 = _tc_layer1(nt2, am2, T, W1,
                             att_src1.reshape(1, HD), att_dst1.reshape(1, HD))
    ex1, dp1 = _sc_pass1(ac1.reshape(N * 2 * H), src3, dst3, z1)
    attn1 = _sc_pass2(ex1.reshape(ETP * H), dp1.reshape(NC, H * NP), dstp)
    og1 = _sc_pass3(hg1.reshape(NG * N, GW), attn1, srcg4, dst3, z128)

    x1, hg2, ac2 = _tc_layer2(x, og1, b1.reshape(1, D), ln1_g.reshape(1, D),
                              ln1_b.reshape(1, D), W2,
                              att_src2.reshape(1, HD), att_dst2.reshape(1, HD))
    ex2, dp2 = _sc_pass1(ac2.reshape(N * 2 * H), src3, dst3, z1)
    attn2 = _sc_pass2(ex2.reshape(ETP * H), dp2.reshape(NC, H * NP), dstp)
    og2 = _sc_pass3(hg2.reshape(NG * N, GW), attn2, srcg4, dst3, z128)

    x2, n2, gacc, macc = _tc_final(x1, og2, b2.reshape(1, D),
                                   ln2_g.reshape(1, D), ln2_b.reshape(1, D),
                                   am2)
    del x2
    ss = _sc_opnorm(n2.reshape(N), ops2)
    hl, osc = _tc_head(gacc, macc, ss.reshape(NOP // 128, 128),
                       rW1, rb1.reshape(1, D), rW2, rb2.reshape(1, D),
                       ln3_g.reshape(1, D), ln3_b.reshape(1, D))
    return hl.reshape(D), osc.reshape(NOP), attn2.reshape(ETP, H)[:ET]


# speedup vs baseline: 12.1215x; 1.0038x over previous
import functools

import jax
import jax.numpy as jnp
from jax import lax
from jax.experimental import pallas as pl
from jax.experimental.pallas import tpu as pltpu
from jax.experimental.pallas import tpu_sc as plsc

N = 10000
E = 160000
H = 4
D = 256
HD = H * D
NG = 8
GW = 128
NOP = 2048
ET = E + N
NC = 2
NS = 16
NW = NC * NS
SUB = 128
NSUB = 42
EC = SUB * NSUB
ETP = EC * NW
SUBP = 64
NSUBP = EC // SUBP
BN = 80
NB = N // BN
NP = 10240
RPT = NP // NS

_f32 = jnp.float32
_i32 = jnp.int32


def _tail_matmul(xb, W_ref, asf_ref, adf_ref, hg_ref, ac_ref):
    hb = jnp.dot(xb, W_ref[...], preferred_element_type=_f32)
    for g in range(NG):
        hg_ref[g] = hb[:, g * GW:(g + 1) * GW]
    cols = []
    for hh in range(H):
        sl = hb[:, hh * D:(hh + 1) * D]
        cols.append(jnp.sum(sl * asf_ref[0, hh * D:(hh + 1) * D][None, :],
                            axis=1, keepdims=True))
    for hh in range(H):
        sl = hb[:, hh * D:(hh + 1) * D]
        cols.append(jnp.sum(sl * adf_ref[0, hh * D:(hh + 1) * D][None, :],
                            axis=1, keepdims=True))
    ac_ref[...] = jnp.concatenate(cols, axis=1)


def _tc1_body(nt_ref, am_ref, T_ref, W_ref, asf_ref, adf_ref,
              x_ref, hg_ref, ac_ref):
    col = lax.broadcasted_iota(_i32, (BN, 128), 1)
    oh = (col == nt_ref[...]).astype(_f32)
    oh = oh + (col == (am_ref[...].astype(_i32) + 3)).astype(_f32)
    xb = jnp.dot(oh, T_ref[...], preferred_element_type=_f32)
    x_ref[...] = xb
    _tail_matmul(xb, W_ref, asf_ref, adf_ref, hg_ref, ac_ref)


def _mix_layer(xprev, og_ref, b_ref, g_ref, be_ref):
    og = og_ref[0] + og_ref[1]
    m0 = (og[0] + og[2] + og[4] + og[6]) * 0.25
    m1 = (og[1] + og[3] + og[5] + og[7]) * 0.25
    hmean = jnp.concatenate([m0, m1], axis=1) + b_ref[...]
    mu = jnp.mean(hmean, axis=1, keepdims=True)
    var = jnp.mean((hmean - mu) ** 2, axis=1, keepdims=True)
    ln = (hmean - mu) * lax.rsqrt(var + 1e-5) * g_ref[...] + be_ref[...]
    return xprev + jnp.maximum(ln, 0.0)


def _tc2_body(xp_ref, og_ref, b_ref, g_ref, be_ref, W_ref, asf_ref, adf_ref,
              x_ref, hg_ref, ac_ref):
    xb = _mix_layer(xp_ref[...], og_ref, b_ref, g_ref, be_ref)
    x_ref[...] = xb
    _tail_matmul(xb, W_ref, asf_ref, adf_ref, hg_ref, ac_ref)


def _tc3_body(xp_ref, og_ref, b_ref, g_ref, be_ref, am_ref,
              x_ref, n2_ref, gacc_ref, macc_ref):
    i = pl.program_id(0)
    xb = _mix_layer(xp_ref[...], og_ref, b_ref, g_ref, be_ref)
    x_ref[...] = xb
    n2_ref[...] = jnp.sum(xb * xb, axis=1, keepdims=True)
    amb = am_ref[...]

    @pl.when(i == 0)
    def _():
        gacc_ref[...] = jnp.zeros_like(gacc_ref)
        macc_ref[...] = jnp.zeros_like(macc_ref)

    gacc_ref[...] += jnp.sum(xb * amb, axis=0, keepdims=True)
    macc_ref[...] += jnp.sum(amb, axis=0, keepdims=True)


def _tc4_body(gacc_ref, macc_ref, ss_ref, rW1_ref, rb1_ref, rW2_ref, rb2_ref,
              g3_ref, b3_ref, hl_ref, os_ref):
    g = gacc_ref[...] / (macc_ref[0, 0] + 1e-8)
    hid = jnp.maximum(
        jnp.dot(g, rW1_ref[...], preferred_element_type=_f32) + rb1_ref[...],
        0.0)
    hid = jnp.dot(hid, rW2_ref[...], preferred_element_type=_f32) + rb2_ref[...]
    mu = jnp.mean(hid, axis=1, keepdims=True)
    var = jnp.mean((hid - mu) ** 2, axis=1, keepdims=True)
    hl_ref[...] = ((hid - mu) * lax.rsqrt(var + 1e-5) * g3_ref[...]
                   + b3_ref[...])
    os_ref[...] = jnp.sqrt(ss_ref[...])


def _tc_layer1(nt2, am2, T, W1, asf, adf):
    full = lambda s: pl.BlockSpec(s, lambda i: (0,) * len(s))
    return pl.pallas_call(
        _tc1_body,
        grid=(NB,),
        in_specs=[
            pl.BlockSpec((BN, 1), lambda i: (i, 0)),
            pl.BlockSpec((BN, 1), lambda i: (i, 0)),
            full((128, D)), full((D, HD)), full((1, HD)), full((1, HD)),
        ],
        out_specs=[
            pl.BlockSpec((BN, D), lambda i: (i, 0)),
            pl.BlockSpec((NG, BN, GW), lambda i: (0, i, 0)),
            pl.BlockSpec((BN, 2 * H), lambda i: (i, 0)),
        ],
        out_shape=[
            jax.ShapeDtypeStruct((N, D), _f32),
            jax.ShapeDtypeStruct((NG, N, GW), _f32),
            jax.ShapeDtypeStruct((N, 2 * H), _f32),
        ],
    )(nt2, am2, T, W1, asf, adf)


def _tc_layer2(xp, og, b, g, be, W2, asf, adf):
    full = lambda s: pl.BlockSpec(s, lambda i: (0,) * len(s))
    return pl.pallas_call(
        _tc2_body,
        grid=(NB,),
        in_specs=[
            pl.BlockSpec((BN, D), lambda i: (i, 0)),
            pl.BlockSpec((NC, NG, BN, GW), lambda i: (0, 0, i, 0)),
            full((1, D)), full((1, D)), full((1, D)),
            full((D, HD)), full((1, HD)), full((1, HD)),
        ],
        out_specs=[
            pl.BlockSpec((BN, D), lambda i: (i, 0)),
            pl.BlockSpec((NG, BN, GW), lambda i: (0, i, 0)),
            pl.BlockSpec((BN, 2 * H), lambda i: (i, 0)),
        ],
        out_shape=[
            jax.ShapeDtypeStruct((N, D), _f32),
            jax.ShapeDtypeStruct((NG, N, GW), _f32),
            jax.ShapeDtypeStruct((N, 2 * H), _f32),
        ],
    )(xp, og, b, g, be, W2, asf, adf)


def _tc_final(xp, og, b, g, be, am2):
    full = lambda s: pl.BlockSpec(s, lambda i: (0,) * len(s))
    return pl.pallas_call(
        _tc3_body,
        grid=(NB,),
        in_specs=[
            pl.BlockSpec((BN, D), lambda i: (i, 0)),
            pl.BlockSpec((NC, NG, BN, GW), lambda i: (0, 0, i, 0)),
            full((1, D)), full((1, D)), full((1, D)),
            pl.BlockSpec((BN, 1), lambda i: (i, 0)),
        ],
        out_specs=[
            pl.BlockSpec((BN, D), lambda i: (i, 0)),
            pl.BlockSpec((BN, 1), lambda i: (i, 0)),
            pl.BlockSpec((1, D), lambda i: (0, 0)),
            pl.BlockSpec((1, 1), lambda i: (0, 0)),
        ],
        out_shape=[
            jax.ShapeDtypeStruct((N, D), _f32),
            jax.ShapeDtypeStruct((N, 1), _f32),
            jax.ShapeDtypeStruct((1, D), _f32),
            jax.ShapeDtypeStruct((1, 1), _f32),
        ],
    )(xp, og, b, g, be, am2)


def _tc_head(gacc, macc, ss2, rW1, rb1, rW2, rb2, g3, b3):
    full = lambda s: pl.BlockSpec(s, lambda: (0,) * len(s))
    return pl.pallas_call(
        _tc4_body,
        in_specs=[
            full((1, D)), full((1, 1)), full((NOP // 128, 128)),
            full((D, D)), full((1, D)), full((D, D)), full((1, D)),
            full((1, D)), full((1, D)),
        ],
        out_specs=[full((1, D)), full((NOP // 128, 128))],
        out_shape=[
            jax.ShapeDtypeStruct((1, D), _f32),
            jax.ShapeDtypeStruct((NOP // 128, 128), _f32),
        ],
    )(gacc, macc, ss2, rW1, rb1, rW2, rb2, g3, b3)


_MESH = dict(core_axis_name="c", subcore_axis_name="s")
_SC_PARAMS = dict(
    mesh=plsc.VectorSubcoreMesh(**_MESH),
    compiler_params=pltpu.CompilerParams(needs_layout_passes=False),
)


def _sc_pass1(attc_f, src3, dst3, z1):
    @functools.partial(
        pl.kernel,
        out_type=[
            jax.ShapeDtypeStruct((NW, NSUB, H, SUB), _f32),
            jax.ShapeDtypeStruct((NC, H, NP), _f32),
        ],
        scratch_types=[
            pltpu.VMEM((N * 2 * H,), _f32),
            pltpu.VMEM((NSUB, SUB), _i32),
            pltpu.VMEM((NSUB, SUB), _i32),
            pltpu.VMEM((H, SUB), _f32),
            pltpu.VMEM_SHARED((NP,), _f32),
            pltpu.VMEM_SHARED((NP,), _f32),
            pltpu.VMEM_SHARED((NP,), _f32),
            pltpu.VMEM_SHARED((NP,), _f32),
        ],
        **_SC_PARAMS,
    )
    def k(attc_hbm, src_hbm, dst_hbm, z1_hbm, ex_hbm, dpart_hbm,
          att_v, src_v, dst_v, ex_v, den0, den1, den2, den3):
        dens = (den0, den1, den2, den3)
        cid = lax.axis_index("c")
        sid = lax.axis_index("s")
        wid = sid * NC + cid
        pltpu.sync_copy(attc_hbm, att_v)
        pltpu.sync_copy(src_hbm.at[wid], src_v)
        pltpu.sync_copy(dst_hbm.at[wid], dst_v)
        r0 = sid * RPT
        for hh in range(H):
            pltpu.sync_copy(z1_hbm.at[pl.ds(r0, RPT)],
                            dens[hh].at[pl.ds(r0, RPT)])
        plsc.subcore_barrier()
        lanes = lax.iota(_i32, 16)

        def sub(jj, carry):
            for k8 in range(SUB // 16):
                sv = src_v[jj, pl.ds(k8 * 16, 16)]
                dv = dst_v[jj, pl.ds(k8 * 16, 16)]
                gid = wid * EC + jj * SUB + k8 * 16 + lanes
                valid = gid < ET
                for hh in range(H):
                    a1 = plsc.load_gather(att_v, [sv * (2 * H) + hh])
                    a2 = plsc.load_gather(att_v, [dv * (2 * H) + (H + hh)])
                    al = a1 + a2
                    al = jnp.where(al >= 0.0, al, 0.2 * al)
                    exv = jnp.where(valid, jnp.exp(al), 0.0)
                    ex_v[hh, pl.ds(k8 * 16, 16)] = exv
            pltpu.sync_copy(ex_v, ex_hbm.at[wid, jj])
            for hh in range(H):
                pltpu.sync_copy(ex_v.at[hh], dens[hh].at[dst_v.at[jj]],
                                add=True)
            return carry

        lax.fori_loop(0, NSUB, sub, 0)
        plsc.subcore_barrier()
        for hh in range(H):
            pltpu.sync_copy(dens[hh].at[pl.ds(r0, RPT)],
                            dpart_hbm.at[cid, hh, pl.ds(r0, RPT)])

    return k(attc_f, src3, dst3, z1)


def _sc_pass2(exf, dpart2, dstf):
    @functools.partial(
        pl.kernel,
        out_type=jax.ShapeDtypeStruct((ETP * H,), _f32),
        scratch_types=[
            pltpu.VMEM((H * NP,), _f32),
            pltpu.VMEM((H * NP,), _f32),
            pltpu.VMEM((EC,), _i32),
            pltpu.VMEM((H * SUB,), _f32),
            pltpu.VMEM((H * SUB,), _f32),
        ],
        **_SC_PARAMS,
    )
    def k(ex_hbm, dpart_hbm, dst_hbm, attn_hbm, dA, dB, dst_f, ex_v, at_v):
        cid = lax.axis_index("c")
        sid = lax.axis_index("s")
        wid = sid * NC + cid
        pltpu.sync_copy(dpart_hbm.at[0], dA)
        pltpu.sync_copy(dpart_hbm.at[1], dB)
        pltpu.sync_copy(dst_hbm.at[pl.ds(wid * EC, EC)], dst_f)
        lanes = lax.iota(_i32, 16)
        lq = lanes // 4
        lr = lanes - lq * 4

        def sub(jj, carry):
            pltpu.sync_copy(ex_hbm.at[pl.ds((wid * NSUB + jj) * H * SUB,
                                            H * SUB)], ex_v)
            for grp in range(SUB * H // 16):
                e0 = grp * 4
                dv = plsc.load_gather(dst_f, [jj * SUB + e0 + lq])
                di = lr * NP + dv
                den = plsc.load_gather(dA, [di]) + plsc.load_gather(dB, [di])
                exg = plsc.load_gather(ex_v, [lr * SUB + e0 + lq])
                at_v[pl.ds(grp * 16, 16)] = exg / (den + 1e-16)
            pltpu.sync_copy(
                at_v, attn_hbm.at[pl.ds((wid * EC + jj * SUB) * H, H * SUB)])
            return carry

        lax.fori_loop(0, NSUB, sub, 0)

    return k(exf, dpart2, dstf)


def _sc_pass3(hflat, attnf, srcg4, dst3b, z128):
    @functools.partial(
        pl.kernel,
        out_type=jax.ShapeDtypeStruct((NC, NG, NP, GW), _f32),
        scratch_types=[
            pltpu.VMEM((NSUBP, SUBP), _i32),
            pltpu.VMEM((NSUBP, SUBP), _i32),
            pltpu.VMEM((H * SUBP,), _f32),
            pltpu.VMEM((H * SUBP,), _f32),
            pltpu.VMEM((H * SUBP,), _f32),
            pltpu.VMEM((SUBP, GW), _f32),
            pltpu.VMEM((SUBP, GW), _f32),
            pltpu.VMEM((SUBP, GW), _f32),
            pltpu.VMEM_SHARED((NP, GW), _f32),
            pltpu.SemaphoreType.DMA,
            pltpu.SemaphoreType.DMA,
            pltpu.SemaphoreType.DMA,
            pltpu.SemaphoreType.DMA,
            pltpu.SemaphoreType.DMA,
            pltpu.SemaphoreType.DMA,
            pltpu.SemaphoreType.DMA,
            pltpu.SemaphoreType.DMA,
            pltpu.SemaphoreType.DMA,
        ],
        **_SC_PARAMS,
    )
    def k(h_hbm, attn_hbm, srcg_hbm, dst_hbm, z_hbm, og_hbm,
          src_v, dst_v, at0, at1, at2, rb0, rb1, rb2, acc_sp,
          sg0, sg1, sg2, ss0, ss1, ss2, sa0, sa1, sa2):
        ats = (at0, at1, at2)
        rbs = (rb0, rb1, rb2)
        sgs = (sg0, sg1, sg2)
        sss = (ss0, ss1, ss2)
        sas = (sa0, sa1, sa2)
        cid = lax.axis_index("c")
        sid = lax.axis_index("s")
        wid = sid * NC + cid
        pltpu.sync_copy(dst_hbm.at[wid], dst_v)
        r0 = sid * RPT

        def at_slice(jj):
            return attn_hbm.at[pl.ds((wid * EC + jj * SUBP) * H, H * SUBP)]

        for g in range(NG):
            hcol = g // 2
            pltpu.sync_copy(srcg_hbm.at[g, wid], src_v)
            pltpu.sync_copy(z_hbm.at[pl.ds(r0, RPT)],
                            acc_sp.at[pl.ds(r0, RPT)])
            plsc.subcore_barrier()
            pltpu.async_copy(h_hbm.at[src_v.at[0]], rbs[0], sgs[0])
            pltpu.async_copy(at_slice(0), ats[0], sas[0])

            def step(st, carry):
                for b in range(3):
                    jj = st * 3 + b
                    bn = (b + 1) % 3
                    pltpu.make_async_copy(
                        h_hbm.at[src_v.at[jj]], rbs[b], sgs[b]).wait()
                    pltpu.make_async_copy(at_slice(jj), ats[b], sas[b]).wait()

                    @pl.when(jj >= 2)
                    def _():
                        pltpu.make_async_copy(
                            rbs[bn], acc_sp.at[dst_v.at[jj]], sss[bn]).wait()

                    @pl.when(jj + 1 < NSUBP)
                    def _():
                        pltpu.async_copy(
                            h_hbm.at[src_v.at[jj + 1]], rbs[bn], sgs[bn])
                        pltpu.async_copy(at_slice(jj + 1), ats[bn], sas[bn])

                    rbuf = rbs[b]
                    at_v = ats[b]

                    def scale16(j8, c2):
                        for e in range(16):
                            eg = j8 * 16 + e
                            sp = plsc.load_gather(
                                at_v,
                                [jnp.broadcast_to(eg * H + hcol, (16,))])
                            for q in range(GW // 16):
                                rbuf[eg, pl.ds(q * 16, 16)] = (
                                    rbuf[eg, pl.ds(q * 16, 16)] * sp)
                        return c2

                    lax.fori_loop(0, SUBP // 16, scale16, 0)
                    pltpu.async_copy(rbs[b], acc_sp.at[dst_v.at[jj]], sss[b],
                                     add=True)
                return carry

            lax.fori_loop(0, NSUBP // 3, step, 0)
            pltpu.make_async_copy(
                rbs[1], acc_sp.at[dst_v.at[0]], sss[1]).wait()
            pltpu.make_async_copy(
                rbs[2], acc_sp.at[dst_v.at[0]], sss[2]).wait()
            plsc.subcore_barrier()
            pltpu.sync_copy(acc_sp.at[pl.ds(r0, RPT)],
                            og_hbm.at[cid, g, pl.ds(r0, RPT)])
            plsc.subcore_barrier()

    return k(hflat, attnf, srcg4, dst3b, z128)


def _sc_opnorm(n2, ops2):
    @functools.partial(
        pl.kernel,
        out_type=jax.ShapeDtypeStruct((NOP,), _f32),
        scratch_types=[
            pltpu.VMEM((N,), _f32),
            pltpu.VMEM((NOP // NW,), _i32),
            pltpu.VMEM((NOP // NW,), _f32),
        ],
        **_SC_PARAMS,
    )
    def k(n2_hbm, ops_hbm, ss_hbm, n2_v, idx_v, s_v):
        cid = lax.axis_index("c")
        sid = lax.axis_index("s")
        wid = sid * NC + cid
        npt = NOP // NW
        pltpu.sync_copy(n2_hbm, n2_v)
        pltpu.sync_copy(ops_hbm.at[wid], idx_v)
        for grp in range(npt // 16):
            iv = idx_v[pl.ds(grp * 16, 16)]
            s_v[pl.ds(grp * 16, 16)] = plsc.load_gather(n2_v, [iv])
        pltpu.sync_copy(s_v, ss_hbm.at[pl.ds(wid * npt, npt)])

    return k(n2, ops2)


def kernel(active_mask, edge_index, node_types, operator_indices, type_emb,
           act_emb, W1, att_src1, att_dst1, b1, ln1_g, ln1_b, W2, att_src2,
           att_dst2, b2, ln2_g, ln2_b, rW1, rb1, rW2, rb2, ln3_g, ln3_b):
    loop = jnp.arange(N, dtype=edge_index.dtype)
    src = jnp.concatenate([edge_index[0], loop])
    dst = jnp.concatenate([edge_index[1], loop])
    pad = jnp.zeros((ETP - ET,), edge_index.dtype)
    srcp = jnp.concatenate([src, pad]).astype(_i32)
    dstp = jnp.concatenate([dst, pad]).astype(_i32)
    src3 = srcp.reshape(NW, NSUB, SUB)
    dst3 = dstp.reshape(NW, NSUB, SUB)
    srcp3 = srcp.reshape(NW, NSUBP, SUBP)
    srcg4 = srcp3[None] + jnp.arange(NG, dtype=_i32)[:, None, None, None] * N
    dst3b = dstp.reshape(NW, NSUBP, SUBP)
    T = jnp.concatenate([type_emb, act_emb,
                         jnp.zeros((128 - 5, D), _f32)], axis=0)
    nt2 = node_types.astype(_i32).reshape(N, 1)
    am2 = active_mask.reshape(N, 1)
    z1 = jnp.zeros((NP,), _f32)
    z128 = jnp.zeros((NP, GW), _f32)
    ops2 = operator_indices.astype(_i32).reshape(NW, NOP // NW)

    x, hg1, ac1 = _tc_layer1(nt2, am2, T, W1,
                             att_src1.reshape(1, HD), att_dst1.reshape(1, HD))
    ex1, dp1 = _sc_pass1(ac1.reshape(N * 2 * H), src3, dst3, z1)
    attn1 = _sc_pass2(ex1.reshape(ETP * H), dp1.reshape(NC, H * NP), dstp)
    og1 = _sc_pass3(hg1.reshape(NG * N, GW), attn1, srcg4, dst3b, z128)

    x1, hg2, ac2 = _tc_layer2(x, og1, b1.reshape(1, D), ln1_g.reshape(1, D),
                              ln1_b.reshape(1, D), W2,
                              att_src2.reshape(1, HD), att_dst2.reshape(1, HD))
    ex2, dp2 = _sc_pass1(ac2.reshape(N * 2 * H), src3, dst3, z1)
    attn2 = _sc_pass2(ex2.reshape(ETP * H), dp2.reshape(NC, H * NP), dstp)
    og2 = _sc_pass3(hg2.reshape(NG * N, GW), attn2, srcg4, dst3b, z128)

    x2, n2, gacc, macc = _tc_final(x1, og2, b2.reshape(1, D),
                                   ln2_g.reshape(1, D), ln2_b.reshape(1, D),
                                   am2)
    del x2
    ss = _sc_opnorm(n2.reshape(N), ops2)
    hl, osc = _tc_head(gacc, macc, ss.reshape(NOP // 128, 128),
                       rW1, rb1.reshape(1, D), rW2, rb2.reshape(1, D),
                       ln3_g.reshape(1, D), ln3_b.reshape(1, D))
    return hl.reshape(D), osc.reshape(NOP), attn2.reshape(ETP, H)[:ET]
